# baseline (speedup 1.0000x reference)
import jax
import jax.numpy as jnp
from jax import lax
from jax.experimental import pallas as pl
from jax.experimental.pallas import tpu as pltpu
from jax.experimental.pallas import tpu_sc as plsc

B = 16
H = 256
W = 256
OH = 512
OW = 512
N = OH * OW
NW = 32
PW = N // (NW // B)
CHUNK = 8192
ROWS = CHUNK // OW
NCH = PW // CHUNK
L = 16
KPR = OW // L


def _interp_body(y_hbm, xt_hbm, out_hbm, y_v, x0_v, x1_v, out_v,
                 y_sem, in_sem0, in_sem1, out_sem0, out_sem1):
    in_sems = (in_sem0, in_sem1)
    out_sems = (out_sem0, out_sem1)
    nc = 2
    wid = lax.axis_index("s") * nc + lax.axis_index("c")
    b = wid // 2
    half = wid % 2
    base_pt = half * PW
    base_row = half * (PW // OW)

    def start_in(ci, s):
        pt0 = base_pt + ci * CHUNK
        pltpu.async_copy(
            xt_hbm.at[0, b, pl.ds(pt0, CHUNK)], x0_v.at[s], in_sems[s])
        pltpu.async_copy(
            xt_hbm.at[1, b, pl.ds(pt0, CHUNK)], x1_v.at[s], in_sems[s])

    def wait_in(ci, s):
        pt0 = base_pt + ci * CHUNK
        pltpu.make_async_copy(
            xt_hbm.at[0, b, pl.ds(pt0, CHUNK)], x0_v.at[s], in_sems[s]).wait()
        pltpu.make_async_copy(
            xt_hbm.at[1, b, pl.ds(pt0, CHUNK)], x1_v.at[s], in_sems[s]).wait()

    def drain_out(s):
        pltpu.make_async_copy(
            out_v.at[s], out_hbm.at[b, pl.ds(base_row, ROWS)],
            out_sems[s]).wait()

    ycp = pltpu.async_copy(y_hbm.at[b], y_v, y_sem)
    start_in(0, 0)
    start_in(1, 1)
    ycp.wait()

    def outer(g, carry):
        for s in range(2):
            ci = 2 * g + s
            row0 = base_row + ci * ROWS
            wait_in(ci, s)

            @pl.when(ci >= 2)
            def _():
                drain_out(s)

            def row_loop(r, rcarry):
                @plsc.parallel_loop(0, KPR, step=1, unroll=8)
                def vec_body(k):
                    kk = r * KPR + k
                    c0 = x0_v[s, pl.ds(kk * L, L)]
                    c1 = x1_v[s, pl.ds(kk * L, L)]
                    r0 = c0 * jnp.float32(H - 1)
                    r1 = c1 * jnp.float32(W - 1)
                    i0 = r0.astype(jnp.int32)
                    i1 = r1.astype(jnp.int32)
                    f0 = r0 - i0.astype(jnp.float32)
                    f1 = r1 - i1.astype(jnp.float32)
                    j0 = jnp.minimum(i0 + 1, H - 1)
                    j1 = jnp.minimum(i1 + 1, W - 1)
                    a0 = i0 << 8
                    a1 = j0 << 8
                    v00 = plsc.load_gather(y_v, [a0 + i1])
                    v01 = plsc.load_gather(y_v, [a0 + j1])
                    v10 = plsc.load_gather(y_v, [a1 + i1])
                    v11 = plsc.load_gather(y_v, [a1 + j1])
                    lo = v00 + (v10 - v00) * f0
                    hi = v01 + (v11 - v01) * f0
                    res = lo + (hi - lo) * f1
                    out_v[s, r, pl.ds(k * L, L)] = res
                return rcarry

            lax.fori_loop(0, ROWS, row_loop, 0)

            pltpu.async_copy(
                out_v.at[s], out_hbm.at[b, pl.ds(row0, ROWS)], out_sems[s])

            @pl.when(ci + 2 < NCH)
            def _():
                start_in(ci + 2, s)
        return carry

    lax.fori_loop(0, NCH // 2, outer, 0)
    for s in range(2):
        drain_out(s)


@jax.jit
def kernel(y, xnew):
    y2 = y.reshape(B, H * W)
    one = lax.optimization_barrier(jnp.ones((), jnp.float32))
    xt = jnp.moveaxis(xnew, -1, 0) * one
    mesh = plsc.VectorSubcoreMesh(core_axis_name="c", subcore_axis_name="s")
    out = pl.kernel(
        _interp_body,
        out_type=jax.ShapeDtypeStruct((B, OH, OW), jnp.float32),
        mesh=mesh,
        compiler_params=pltpu.CompilerParams(needs_layout_passes=False),
        scratch_types=[
            pltpu.VMEM((H * W,), jnp.float32),
            pltpu.VMEM((2, CHUNK), jnp.float32),
            pltpu.VMEM((2, CHUNK), jnp.float32),
            pltpu.VMEM((2, ROWS, OW), jnp.float32),
            pltpu.SemaphoreType.DMA,
            pltpu.SemaphoreType.DMA,
            pltpu.SemaphoreType.DMA,
            pltpu.SemaphoreType.DMA,
            pltpu.SemaphoreType.DMA,
        ],
    )(y2, xt)
    return out


# speedup vs baseline: 1.1789x; 1.1789x over previous
import jax
import jax.numpy as jnp
from jax import lax
from jax.experimental import pallas as pl
from jax.experimental.pallas import tpu as pltpu
from jax.experimental.pallas import tpu_sc as plsc

B = 16
H = 256
W = 256
OH = 512
OW = 512
N = OH * OW
NW = 32
PW = N // (NW // B)
CHUNK = 8192
ROWS = CHUNK // OW
NCH = PW // CHUNK
L = 16
KPR = OW // L


def _interp_body(y_hbm, xt_hbm, out_hbm, y_v, x0_v, x1_v, out_v,
                 y_sem, in_sem0, in_sem1, out_sem0, out_sem1):
    in_sems = (in_sem0, in_sem1)
    out_sems = (out_sem0, out_sem1)
    nc = 2
    wid = lax.axis_index("s") * nc + lax.axis_index("c")
    b = wid // 2
    half = wid % 2
    base_pt = half * PW
    base_row = half * (PW // OW)

    def start_in(ci, s):
        pt0 = base_pt + ci * CHUNK
        pltpu.async_copy(
            xt_hbm.at[0, b, pl.ds(pt0, CHUNK)], x0_v.at[s], in_sems[s])
        pltpu.async_copy(
            xt_hbm.at[1, b, pl.ds(pt0, CHUNK)], x1_v.at[s], in_sems[s])

    def wait_in(ci, s):
        pt0 = base_pt + ci * CHUNK
        pltpu.make_async_copy(
            xt_hbm.at[0, b, pl.ds(pt0, CHUNK)], x0_v.at[s], in_sems[s]).wait()
        pltpu.make_async_copy(
            xt_hbm.at[1, b, pl.ds(pt0, CHUNK)], x1_v.at[s], in_sems[s]).wait()

    def drain_out(s):
        pltpu.make_async_copy(
            out_v.at[s], out_hbm.at[b, pl.ds(base_row, ROWS)],
            out_sems[s]).wait()

    ycp = pltpu.async_copy(y_hbm.at[b], y_v, y_sem)
    start_in(0, 0)
    start_in(1, 1)
    ycp.wait()

    def outer(g, carry):
        for s in range(2):
            ci = 2 * g + s
            row0 = base_row + ci * ROWS
            wait_in(ci, s)

            @pl.when(ci >= 2)
            def _():
                drain_out(s)

            def row_loop(r, rcarry):
                @plsc.parallel_loop(0, KPR, step=1, unroll=16)
                def vec_body(k):
                    kk = r * KPR + k
                    c0 = x0_v[s, pl.ds(kk * L, L)]
                    c1 = x1_v[s, pl.ds(kk * L, L)]
                    r0 = c0 * jnp.float32(H - 1)
                    r1 = c1 * jnp.float32(W - 1)
                    i0 = r0.astype(jnp.int32)
                    i1 = r1.astype(jnp.int32)
                    f0 = r0 - i0.astype(jnp.float32)
                    f1 = r1 - i1.astype(jnp.float32)
                    j0 = jnp.minimum(i0 + 1, H - 1)
                    j1 = jnp.minimum(i1 + 1, W - 1)
                    a0 = i0 << 8
                    a1 = j0 << 8
                    v00 = plsc.load_gather(y_v, [a0 + i1])
                    v01 = plsc.load_gather(y_v, [a0 + j1])
                    v10 = plsc.load_gather(y_v, [a1 + i1])
                    v11 = plsc.load_gather(y_v, [a1 + j1])
                    lo = v00 + (v10 - v00) * f0
                    hi = v01 + (v11 - v01) * f0
                    res = lo + (hi - lo) * f1
                    out_v[s, r, pl.ds(k * L, L)] = res
                return rcarry

            lax.fori_loop(0, ROWS, row_loop, 0)

            pltpu.async_copy(
                out_v.at[s], out_hbm.at[b, pl.ds(row0, ROWS)], out_sems[s])

            @pl.when(ci + 2 < NCH)
            def _():
                start_in(ci + 2, s)
        return carry

    lax.fori_loop(0, NCH // 2, outer, 0)
    for s in range(2):
        drain_out(s)


@jax.jit
def kernel(y, xnew):
    y2 = y.reshape(B, H * W)
    xt = jnp.moveaxis(xnew, -1, 0)
    mesh = plsc.VectorSubcoreMesh(core_axis_name="c", subcore_axis_name="s")
    out = pl.kernel(
        _interp_body,
        out_type=jax.ShapeDtypeStruct((B, OH, OW), jnp.float32),
        mesh=mesh,
        compiler_params=pltpu.CompilerParams(needs_layout_passes=False),
        scratch_types=[
            pltpu.VMEM((H * W,), jnp.float32),
            pltpu.VMEM((2, CHUNK), jnp.float32),
            pltpu.VMEM((2, CHUNK), jnp.float32),
            pltpu.VMEM((2, ROWS, OW), jnp.float32),
            pltpu.SemaphoreType.DMA,
            pltpu.SemaphoreType.DMA,
            pltpu.SemaphoreType.DMA,
            pltpu.SemaphoreType.DMA,
            pltpu.SemaphoreType.DMA,
        ],
    )(y2, xt)
    return out


# speedup vs baseline: 1.2080x; 1.0247x over previous
import jax
import jax.numpy as jnp
from jax import lax
from jax.experimental import pallas as pl
from jax.experimental.pallas import tpu as pltpu
from jax.experimental.pallas import tpu_sc as plsc

B = 16
H = 256
W = 256
OH = 512
OW = 512
N = OH * OW
NW = 32
PW = N // (NW // B)
CHUNK = 8192
ROWS = CHUNK // OW
NCH = PW // CHUNK
L = 16
KPR = OW // L


def _interp_body(y_hbm, xt_hbm, out_hbm, y_v, x0_v, x1_v, out_v,
                 y_sem, in_sem0, in_sem1, out_sem0, out_sem1):
    in_sems = (in_sem0, in_sem1)
    out_sems = (out_sem0, out_sem1)
    nc = 2
    wid = lax.axis_index("s") * nc + lax.axis_index("c")
    b = wid // 2
    half = wid % 2
    base_pt = half * PW
    base_row = half * (PW // OW)

    def start_in(ci, s):
        pt0 = base_pt + ci * CHUNK
        pltpu.async_copy(
            xt_hbm.at[0, b, pl.ds(pt0, CHUNK)], x0_v.at[s], in_sems[s])
        pltpu.async_copy(
            xt_hbm.at[1, b, pl.ds(pt0, CHUNK)], x1_v.at[s], in_sems[s])

    def wait_in(ci, s):
        pt0 = base_pt + ci * CHUNK
        pltpu.make_async_copy(
            xt_hbm.at[0, b, pl.ds(pt0, CHUNK)], x0_v.at[s], in_sems[s]).wait()
        pltpu.make_async_copy(
            xt_hbm.at[1, b, pl.ds(pt0, CHUNK)], x1_v.at[s], in_sems[s]).wait()

    def drain_out(s):
        pltpu.make_async_copy(
            out_v.at[s], out_hbm.at[b, pl.ds(base_row, ROWS)],
            out_sems[s]).wait()

    ycp = pltpu.async_copy(y_hbm.at[b], y_v, y_sem)
    start_in(0, 0)
    start_in(1, 1)
    ycp.wait()

    def outer(g, carry):
        for s in range(2):
            ci = 2 * g + s
            row0 = base_row + ci * ROWS
            wait_in(ci, s)

            @pl.when(ci >= 2)
            def _():
                drain_out(s)

            def row_loop(r, rcarry):
                @plsc.parallel_loop(0, KPR, step=1, unroll=32)
                def vec_body(k):
                    kk = r * KPR + k
                    c0 = x0_v[s, pl.ds(kk * L, L)]
                    c1 = x1_v[s, pl.ds(kk * L, L)]
                    r0 = c0 * jnp.float32(H - 1)
                    r1 = c1 * jnp.float32(W - 1)
                    i0 = r0.astype(jnp.int32)
                    i1 = r1.astype(jnp.int32)
                    f0 = r0 - i0.astype(jnp.float32)
                    f1 = r1 - i1.astype(jnp.float32)
                    j0 = jnp.minimum(i0 + 1, H - 1)
                    j1 = jnp.minimum(i1 + 1, W - 1)
                    a0 = i0 << 8
                    a1 = j0 << 8
                    v00 = plsc.load_gather(y_v, [a0 + i1])
                    v01 = plsc.load_gather(y_v, [a0 + j1])
                    v10 = plsc.load_gather(y_v, [a1 + i1])
                    v11 = plsc.load_gather(y_v, [a1 + j1])
                    lo = v00 + (v10 - v00) * f0
                    hi = v01 + (v11 - v01) * f0
                    res = lo + (hi - lo) * f1
                    out_v[s, r, pl.ds(k * L, L)] = res
                return rcarry

            lax.fori_loop(0, ROWS, row_loop, 0)

            pltpu.async_copy(
                out_v.at[s], out_hbm.at[b, pl.ds(row0, ROWS)], out_sems[s])

            @pl.when(ci + 2 < NCH)
            def _():
                start_in(ci + 2, s)
        return carry

    lax.fori_loop(0, NCH // 2, outer, 0)
    for s in range(2):
        drain_out(s)


@jax.jit
def kernel(y, xnew):
    y2 = y.reshape(B, H * W)
    xt = jnp.moveaxis(xnew, -1, 0)
    mesh = plsc.VectorSubcoreMesh(core_axis_name="c", subcore_axis_name="s")
    out = pl.kernel(
        _interp_body,
        out_type=jax.ShapeDtypeStruct((B, OH, OW), jnp.float32),
        mesh=mesh,
        compiler_params=pltpu.CompilerParams(needs_layout_passes=False),
        scratch_types=[
            pltpu.VMEM((H * W,), jnp.float32),
            pltpu.VMEM((2, CHUNK), jnp.float32),
            pltpu.VMEM((2, CHUNK), jnp.float32),
            pltpu.VMEM((2, ROWS, OW), jnp.float32),
            pltpu.SemaphoreType.DMA,
            pltpu.SemaphoreType.DMA,
            pltpu.SemaphoreType.DMA,
            pltpu.SemaphoreType.DMA,
            pltpu.SemaphoreType.DMA,
        ],
    )(y2, xt)
    return out
